# baseline (speedup 1.0000x reference)
import jax
import jax.numpy as jnp
from jax import lax
from jax.experimental import pallas as pl
from jax.experimental.pallas import tpu as pltpu
from jax.experimental.pallas import tpu_sc as plsc

_NS = 16
_B, _T, _F = 4096, 200, 128
_FO = _F // 2


def _make_sc_half(b_start, b_count):
  mesh = plsc.VectorSubcoreMesh(
      core_axis_name="c", subcore_axis_name="s",
      num_cores=1, num_subcores=_NS)
  nw = _NS
  b_per_tile = b_count // nw

  def body(x_hbm, out_hbm, in0, in1, out0, out1, si0, si1, so0, so1):
    wid = lax.axis_index("s")
    b0 = b_start + wid * b_per_tile
    ob0 = wid * b_per_tile
    evens = lax.iota(jnp.int32, 16) * 2
    cols = [evens + 32 * q for q in range(_FO // 16)]

    in_bufs = (in0, in1)
    out_bufs = (out0, out1)
    in_sems = (si0, si1)
    out_sems = (so0, so1)

    def issue_in(k, b):
      pltpu.async_copy(x_hbm.at[b0 + k], in_bufs[b], in_sems[b])

    def wait_in(b):
      pltpu.make_async_copy(x_hbm.at[0], in_bufs[b], in_sems[b]).wait()

    def issue_out(k, b):
      pltpu.async_copy(out_bufs[b], out_hbm.at[ob0 + k], out_sems[b])

    def wait_out(b):
      pltpu.make_async_copy(out_bufs[b], out_hbm.at[0], out_sems[b]).wait()

    def compute(b):
      src = in_bufs[b]
      dst = out_bufs[b]

      @plsc.parallel_loop(0, _T, unroll=4)
      def _(r):
        row = jnp.full((16,), r, jnp.int32)
        for q in range(_FO // 16):
          vals = plsc.load_gather(src, [row, cols[q]])
          dst[r, pl.ds(16 * q, 16)] = vals

    issue_in(0, 0)
    issue_in(1, 1)
    for k in (0, 1):
      b = k & 1
      wait_in(b)
      compute(b)
      issue_out(k, b)
      issue_in(k + 2, b)

    @pl.loop(0, (b_per_tile - 4) // 2)
    def _(i):
      for b in (0, 1):
        k = 2 + 2 * i + b
        wait_in(b)
        wait_out(b)
        compute(b)
        issue_out(k, b)
        issue_in(k + 2, b)

    for k in (b_per_tile - 2, b_per_tile - 1):
      b = k & 1
      wait_in(b)
      wait_out(b)
      compute(b)
      issue_out(k, b)
    wait_out(0)
    wait_out(1)

  return pl.kernel(
      body,
      out_type=jax.ShapeDtypeStruct((b_count, _T, _FO), jnp.float32),
      mesh=mesh,
      compiler_params=pltpu.CompilerParams(needs_layout_passes=False),
      scratch_types=[
          pltpu.VMEM((_T, _F), jnp.float32),
          pltpu.VMEM((_T, _F), jnp.float32),
          pltpu.VMEM((_T, _FO), jnp.float32),
          pltpu.VMEM((_T, _FO), jnp.float32),
          pltpu.SemaphoreType.DMA,
          pltpu.SemaphoreType.DMA,
          pltpu.SemaphoreType.DMA,
          pltpu.SemaphoreType.DMA,
      ],
  )


_half_a = _make_sc_half(0, _B // 2)
_half_b = _make_sc_half(_B // 2, _B // 2)


def kernel(x):
  return jnp.concatenate([_half_a(x), _half_b(x)], axis=0)


# speedup vs baseline: 1.4615x; 1.4615x over previous
import jax
import jax.numpy as jnp
from jax import lax
from jax.experimental import pallas as pl
from jax.experimental.pallas import tpu as pltpu
from jax.experimental.pallas import tpu_sc as plsc

_NC = 2
_NS = 16
_NW = _NC * _NS

_B, _T, _F = 4096, 200, 128
_FO = _F // 2
_B_PER_TILE = _B // _NW
_VPR = _FO // 16


def _make_sc_call():
  mesh = plsc.VectorSubcoreMesh(
      core_axis_name="c", subcore_axis_name="s",
      num_cores=_NC, num_subcores=_NS)

  def body(x_hbm, out_hbm, in0, in1, out0, out1, si0, si1, so0, so1):
    wid = lax.axis_index("c") * _NS + lax.axis_index("s")
    b0 = wid * _B_PER_TILE
    evens = lax.iota(jnp.int32, 16) * 2
    cols = [evens + 32 * q for q in range(_VPR)]

    in_bufs = (in0, in1)
    out_bufs = (out0, out1)
    in_sems = (si0, si1)
    out_sems = (so0, so1)

    def issue_in(k, b):
      pltpu.async_copy(x_hbm.at[b0 + k], in_bufs[b], in_sems[b])

    def wait_in(b):
      pltpu.make_async_copy(x_hbm.at[0], in_bufs[b], in_sems[b]).wait()

    def issue_out(k, b):
      pltpu.async_copy(out_bufs[b], out_hbm.at[b0 + k], out_sems[b])

    def wait_out(b):
      pltpu.make_async_copy(out_bufs[b], out_hbm.at[0], out_sems[b]).wait()

    def compute(b):
      src = in_bufs[b]
      dst = out_bufs[b]

      @plsc.parallel_loop(0, _T, unroll=4)
      def _(r):
        row = jnp.full((16,), r, jnp.int32)
        for q in range(_VPR):
          vals = plsc.load_gather(src, [row, cols[q]])
          dst[r, pl.ds(16 * q, 16)] = vals

    issue_in(0, 0)
    issue_in(1, 1)
    for k in (0, 1):
      b = k & 1
      wait_in(b)
      compute(b)
      issue_out(k, b)
      issue_in(k + 2, b)

    @pl.loop(0, (_B_PER_TILE - 4) // 2)
    def _(i):
      for b in (0, 1):
        k = 2 + 2 * i + b
        wait_in(b)
        wait_out(b)
        compute(b)
        issue_out(k, b)
        issue_in(k + 2, b)

    for k in (_B_PER_TILE - 2, _B_PER_TILE - 1):
      b = k & 1
      wait_in(b)
      wait_out(b)
      compute(b)
      issue_out(k, b)
    wait_out(0)
    wait_out(1)

  return pl.kernel(
      body,
      out_type=jax.ShapeDtypeStruct((_B, _T, _FO), jnp.float32),
      mesh=mesh,
      compiler_params=pltpu.CompilerParams(needs_layout_passes=False),
      scratch_types=[
          pltpu.VMEM((_T, _F), jnp.float32),
          pltpu.VMEM((_T, _F), jnp.float32),
          pltpu.VMEM((_T, _FO), jnp.float32),
          pltpu.VMEM((_T, _FO), jnp.float32),
          pltpu.SemaphoreType.DMA,
          pltpu.SemaphoreType.DMA,
          pltpu.SemaphoreType.DMA,
          pltpu.SemaphoreType.DMA,
      ],
  )


_sc_slice = _make_sc_call()


def kernel(x):
  return _sc_slice(x)
